# baseline (speedup 1.0000x reference)
import jax, jax.numpy as jnp

B, C, H, W, A = 4, 256, 40, 40, 9
HW = H * W

def kernel(features, W_conv, b_conv, W_obj, b_obj, W_bbox, b_bbox):
    box_t = (features[:, :36] * 2.0).reshape(B, 36, HW)
    cols = [jnp.transpose(box_t[:, c::4, :], (0, 2, 1)) for c in range(4)]
    bb = jnp.stack(cols, axis=3).reshape(B, HW * A, 4)
    return bb


# speedup vs baseline: 1.6312x; 1.6312x over previous
import numpy as np
import jax
import jax.numpy as jnp
from jax import lax
from jax.experimental import pallas as pl

B, C, H, W, A = 4, 256, 40, 40, 9
HW = H * W
STRIDE = 16
SCALES = (64.0, 128.0, 256.0)
RATIOS = (0.5, 1.0, 2.0)


def _anchors_const():
    xs = (np.arange(W, dtype=np.float32) + 0.5) * STRIDE
    ys = (np.arange(H, dtype=np.float32) + 0.5) * STRIDE
    whs = np.array([(s * np.sqrt(r), s / np.sqrt(r))
                    for s in SCALES for r in RATIOS], dtype=np.float32)
    cx = np.broadcast_to(xs[None, :, None], (H, W, A))
    cy = np.broadcast_to(ys[:, None, None], (H, W, A))
    aw = np.broadcast_to(whs[None, None, :, 0], (H, W, A))
    ah = np.broadcast_to(whs[None, None, :, 1], (H, W, A))
    a = np.stack([cx, cy, aw, ah], axis=-1).reshape(HW * A * 4)
    return a.reshape(1, HW * A * 4 // 128, 128)


_ANCHORS = _anchors_const()
_AR = _ANCHORS.shape[1]


def _rpn_body(x_ref, wt_ref, bc_ref, wo_ref, bo_ref, wb_ref, bb_ref,
              obj_ref, box_ref):
    x = x_ref[0]
    acc = jnp.zeros((HW, C), jnp.float32)
    for k in range(9):
        dy, dx = k // 3, k % 3
        xs = x[dy:dy + H, dx:dx + W, :].reshape(HW, C)
        acc = acc + jnp.dot(xs, wt_ref[k], preferred_element_type=jnp.float32)
    h = jnp.maximum(acc + bc_ref[0], 0.0).astype(jnp.bfloat16)
    obj = lax.dot_general(
        wo_ref[...], h, (((1,), (1,)), ((), ())),
        preferred_element_type=jnp.float32) + bo_ref[...]
    box_t = lax.dot_general(
        wb_ref[...], h, (((1,), (1,)), ((), ())),
        preferred_element_type=jnp.float32) + bb_ref[...]
    box_ref[0] = box_t
    for y in range(H):
        obj_ref[0, :, y, :] = obj[:, y * W:(y + 1) * W]


def kernel(features, W_conv, b_conv, W_obj, b_obj, W_bbox, b_bbox):
    x = jnp.transpose(features, (0, 2, 3, 1))
    xpad = jnp.pad(x, ((0, 0), (1, 1), (1, 1), (0, 0))).astype(jnp.bfloat16)
    wt = jnp.transpose(W_conv, (2, 3, 1, 0)).reshape(9, C, C).astype(jnp.bfloat16)
    wo = W_obj.reshape(A, C).astype(jnp.bfloat16)
    wb = W_bbox.reshape(4 * A, C).astype(jnp.bfloat16)
    bc = b_conv.reshape(1, C)
    bo = b_obj.reshape(A, 1)
    bb = b_bbox.reshape(4 * A, 1)

    obj, box = pl.pallas_call(
        _rpn_body,
        grid=(B,),
        in_specs=[
            pl.BlockSpec((1, H + 2, W + 2, C), lambda b: (b, 0, 0, 0)),
            pl.BlockSpec((9, C, C), lambda b: (0, 0, 0)),
            pl.BlockSpec((1, C), lambda b: (0, 0)),
            pl.BlockSpec((A, C), lambda b: (0, 0)),
            pl.BlockSpec((A, 1), lambda b: (0, 0)),
            pl.BlockSpec((4 * A, C), lambda b: (0, 0)),
            pl.BlockSpec((4 * A, 1), lambda b: (0, 0)),
        ],
        out_specs=[
            pl.BlockSpec((1, A, H, W), lambda b: (b, 0, 0, 0)),
            pl.BlockSpec((1, 4 * A, HW), lambda b: (b, 0, 0)),
        ],
        out_shape=[
            jax.ShapeDtypeStruct((B, A, H, W), jnp.float32),
            jax.ShapeDtypeStruct((B, 4 * A, HW), jnp.float32),
        ],
    )(xpad, wt, bc, wo, bo, wb, bb)

    objness = obj.reshape(B, A * HW, 1)
    bb4 = box.reshape(B, A, 4, HW)
    bb_out = jnp.transpose(bb4, (0, 3, 1, 2)).reshape(B, HW * A, 4)
    anchors = jnp.broadcast_to(
        jnp.asarray(_ANCHORS.reshape(HW * A, 4))[None], (B, HW * A, 4))
    return (objness, bb_out, anchors)


# speedup vs baseline: 1.6431x; 1.0073x over previous
import numpy as np
import jax
import jax.numpy as jnp
from jax import lax
from jax.experimental import pallas as pl

B, C, H, W, A = 4, 256, 40, 40, 9
HW = H * W
STRIDE = 16
SCALES = (64.0, 128.0, 256.0)
RATIOS = (0.5, 1.0, 2.0)


def _anchors_const():
    xs = (np.arange(W, dtype=np.float32) + 0.5) * STRIDE
    ys = (np.arange(H, dtype=np.float32) + 0.5) * STRIDE
    whs = np.array([(s * np.sqrt(r), s / np.sqrt(r))
                    for s in SCALES for r in RATIOS], dtype=np.float32)
    cx = np.broadcast_to(xs[None, :, None], (H, W, A))
    cy = np.broadcast_to(ys[:, None, None], (H, W, A))
    aw = np.broadcast_to(whs[None, None, :, 0], (H, W, A))
    ah = np.broadcast_to(whs[None, None, :, 1], (H, W, A))
    a = np.stack([cx, cy, aw, ah], axis=-1).reshape(HW * A * 4)
    return a.reshape(1, HW * A * 4 // 128, 128)


_ANCHORS = _anchors_const()
_AR = _ANCHORS.shape[1]


def _rpn_body(x_ref, wt_ref, bc_ref, wo_ref, bo_ref, wb_ref, bb_ref,
              obj_ref, box_ref):
    x = x_ref[0]
    acc = jnp.zeros((HW, C), jnp.float32)
    for k in range(9):
        dy, dx = k // 3, k % 3
        xs = x[dy:dy + H, dx:dx + W, :].reshape(HW, C)
        acc = acc + jnp.dot(xs, wt_ref[k], preferred_element_type=jnp.float32)
    h = jnp.maximum(acc + bc_ref[0], 0.0).astype(jnp.bfloat16)
    obj = lax.dot_general(
        wo_ref[...], h, (((1,), (1,)), ((), ())),
        preferred_element_type=jnp.float32) + bo_ref[...]
    box_t = lax.dot_general(
        wb_ref[...], h, (((1,), (1,)), ((), ())),
        preferred_element_type=jnp.float32) + bb_ref[...]
    box_ref[0] = box_t.astype(jnp.bfloat16)
    for y in range(H):
        obj_ref[0, :, y, :] = obj[:, y * W:(y + 1) * W]


def kernel(features, W_conv, b_conv, W_obj, b_obj, W_bbox, b_bbox):
    x = jnp.transpose(features, (0, 2, 3, 1))
    xpad = jnp.pad(x, ((0, 0), (1, 1), (1, 1), (0, 0))).astype(jnp.bfloat16)
    wt = jnp.transpose(W_conv, (2, 3, 1, 0)).reshape(9, C, C).astype(jnp.bfloat16)
    wo = W_obj.reshape(A, C).astype(jnp.bfloat16)
    wb = W_bbox.reshape(4 * A, C).astype(jnp.bfloat16)
    bc = b_conv.reshape(1, C)
    bo = b_obj.reshape(A, 1)
    bb = b_bbox.reshape(4 * A, 1)

    obj, box = pl.pallas_call(
        _rpn_body,
        grid=(B,),
        in_specs=[
            pl.BlockSpec((1, H + 2, W + 2, C), lambda b: (b, 0, 0, 0)),
            pl.BlockSpec((9, C, C), lambda b: (0, 0, 0)),
            pl.BlockSpec((1, C), lambda b: (0, 0)),
            pl.BlockSpec((A, C), lambda b: (0, 0)),
            pl.BlockSpec((A, 1), lambda b: (0, 0)),
            pl.BlockSpec((4 * A, C), lambda b: (0, 0)),
            pl.BlockSpec((4 * A, 1), lambda b: (0, 0)),
        ],
        out_specs=[
            pl.BlockSpec((1, A, H, W), lambda b: (b, 0, 0, 0)),
            pl.BlockSpec((1, 4 * A, HW), lambda b: (b, 0, 0)),
        ],
        out_shape=[
            jax.ShapeDtypeStruct((B, A, H, W), jnp.float32),
            jax.ShapeDtypeStruct((B, 4 * A, HW), jnp.bfloat16),
        ],
    )(xpad, wt, bc, wo, bo, wb, bb)

    objness = obj.reshape(B, A * HW, 1)
    bb4 = box.reshape(B, A, 4, HW)
    bb_out = jnp.transpose(bb4, (0, 3, 1, 2)).reshape(
        B, HW * A, 4).astype(jnp.float32)
    anchors = jnp.broadcast_to(
        jnp.asarray(_ANCHORS.reshape(HW * A, 4))[None], (B, HW * A, 4))
    return (objness, bb_out, anchors)
